# baseline (speedup 1.0000x reference)
import functools

import jax
import jax.numpy as jnp
from jax import lax
from jax.experimental import pallas as pl
from jax.experimental.pallas import tpu as pltpu
from jax.experimental.pallas import tpu_sc as plsc

DIM = 2048
N_EXPERTS = 64
N_GROUPS = 8
GROUP_SIZE = N_EXPERTS // N_GROUPS
TOPK_GROUPS = 4
TOPK = 8
ROUTE_SCALE = 2.5
N_TOK = 16384

BT = 512

L = 16
NW = 32
CHUNKS = (4096, 4096, 4096, 2048, 2048)


NE_PAD = 128


def _score_kernel(x_ref, w_ref, s_ref):
    scores = jax.nn.sigmoid(
        jax.lax.dot_general(
            x_ref[...], w_ref[...],
            dimension_numbers=(((1,), (1,)), ((), ())),
            preferred_element_type=jnp.float32,
        )
    )
    pad = jnp.zeros((scores.shape[0], NE_PAD - N_EXPERTS), jnp.float32)
    s_ref[...] = jnp.concatenate([scores, pad], axis=1)


def _tree_max(vs):
    while len(vs) > 1:
        nxt = [jnp.maximum(vs[i], vs[i + 1]) for i in range(0, len(vs) - 1, 2)]
        if len(vs) % 2:
            nxt.append(vs[-1])
        vs = nxt
    return vs[0]


def _route_kernel(tok_per_w, s_hbm, w_hbm, i_hbm, s_v, w_v, i_v):
    wid = lax.axis_index("s") * 2 + lax.axis_index("c")
    base = wid * tok_per_w
    pltpu.sync_copy(s_hbm.at[pl.ds(base * NE_PAD, tok_per_w * NE_PAD)], s_v)

    lanes = lax.iota(jnp.int32, L)

    def body(i, carry):
        tok = i * L + lanes

        tok64 = tok * NE_PAD
        s = [
            plsc.load_gather(s_v, [tok64 + e])
            for e in range(N_EXPERTS)
        ]

        gm = [
            _tree_max(s[g * GROUP_SIZE:(g + 1) * GROUP_SIZE])
            for g in range(N_GROUPS)
        ]

        neg1 = jnp.full((L,), -1.0, jnp.float32)
        gsel8 = []
        work = list(gm)
        for _q in range(TOPK_GROUPS):
            cur = _tree_max(work)
            gidx = jnp.full((L,), N_GROUPS, jnp.int32)
            for g in range(N_GROUPS):
                gidx = jnp.minimum(
                    gidx,
                    jnp.where(work[g] == cur,
                              jnp.full((L,), g, jnp.int32),
                              jnp.full((L,), N_GROUPS, jnp.int32)),
                )
            for g in range(N_GROUPS):
                work[g] = jnp.where(gidx == g, neg1, work[g])
            gsel8.append(gidx * GROUP_SIZE)

        cands = []
        cols = []
        for q in range(TOPK_GROUPS):
            for j in range(GROUP_SIZE):
                col = gsel8[q] + j
                cols.append(col)
                cands.append(plsc.load_gather(s_v, [tok64 + col]))

        big = jnp.full((L,), N_EXPERTS, jnp.int32)
        ws = []
        idxs = []
        for _k in range(TOPK):
            cur = _tree_max(cands)
            idx = big
            for e in range(len(cands)):
                idx = jnp.minimum(idx, jnp.where(cands[e] == cur, cols[e], big))
            for e in range(len(cands)):
                cands[e] = jnp.where(cols[e] == idx, neg1, cands[e])
            ws.append(cur)
            idxs.append(idx)

        total = (ws[0] + ws[1]) + (ws[2] + ws[3]) + ((ws[4] + ws[5])
                                                    + (ws[6] + ws[7]))
        scale = ROUTE_SCALE / total
        tok8 = tok * TOPK
        for k in range(TOPK):
            plsc.store_scatter(w_v, [tok8 + k], ws[k] * scale)
            plsc.store_scatter(i_v, [tok8 + k], idxs[k])
        return carry

    lax.fori_loop(0, tok_per_w // L, body, 0)

    pltpu.sync_copy(w_v, w_hbm.at[pl.ds(base * TOPK, tok_per_w * TOPK)])
    pltpu.sync_copy(i_v, i_hbm.at[pl.ds(base * TOPK, tok_per_w * TOPK)])


@functools.cache
def _make_route(tokc):
    tok_per_w = tokc // NW
    return pl.kernel(
        functools.partial(_route_kernel, tok_per_w),
        mesh=plsc.VectorSubcoreMesh(core_axis_name="c", subcore_axis_name="s"),
        out_type=[
            jax.ShapeDtypeStruct((tokc * TOPK,), jnp.float32),
            jax.ShapeDtypeStruct((tokc * TOPK,), jnp.int32),
        ],
        scratch_types=[
            pltpu.VMEM((tok_per_w * NE_PAD,), jnp.float32),
            pltpu.VMEM((tok_per_w * TOPK,), jnp.float32),
            pltpu.VMEM((tok_per_w * TOPK,), jnp.int32),
        ],
        compiler_params=pltpu.CompilerParams(needs_layout_passes=False),
    )


@jax.jit
def kernel(x, weight):
    n = x.shape[0]
    w_parts = []
    i_parts = []
    tok_off = 0
    w_dep = weight
    for tokc in CHUNKS:
        off = tok_off // BT
        scores = pl.pallas_call(
            _score_kernel,
            grid=(tokc // BT,),
            in_specs=[
                pl.BlockSpec((BT, DIM), lambda i, off=off: (off + i, 0)),
                pl.BlockSpec((N_EXPERTS, DIM), lambda i: (0, 0)),
            ],
            out_specs=pl.BlockSpec((BT, NE_PAD), lambda i: (i, 0)),
            out_shape=jax.ShapeDtypeStruct((tokc, NE_PAD), jnp.float32),
        )(x, w_dep)
        w_dep = lax.optimization_barrier((weight, scores))[0]
        w_flat, i_flat = _make_route(tokc)(scores.reshape(-1))
        w_parts.append(w_flat.reshape(tokc, TOPK))
        i_parts.append(i_flat.reshape(tokc, TOPK))
        tok_off += tokc
    return (jnp.concatenate(w_parts, axis=0),
            jnp.concatenate(i_parts, axis=0))


# speedup vs baseline: 1.1908x; 1.1908x over previous
import functools

import jax
import jax.numpy as jnp
from jax import lax
from jax.experimental import pallas as pl
from jax.experimental.pallas import tpu as pltpu
from jax.experimental.pallas import tpu_sc as plsc

DIM = 2048
N_EXPERTS = 64
N_GROUPS = 8
GROUP_SIZE = N_EXPERTS // N_GROUPS
TOPK_GROUPS = 4
TOPK = 8
ROUTE_SCALE = 2.5
N_TOK = 16384

BT = 512

L = 16
NW = 32
CHUNKS = (4096, 4096, 4096, 4096)


NE_PAD = 128


def _score_kernel(x_ref, w_ref, s_ref):
    scores = jax.nn.sigmoid(
        jax.lax.dot_general(
            x_ref[...], w_ref[...],
            dimension_numbers=(((1,), (1,)), ((), ())),
            preferred_element_type=jnp.float32,
        )
    )
    pad = jnp.zeros((scores.shape[0], NE_PAD - N_EXPERTS), jnp.float32)
    s_ref[...] = jnp.concatenate([scores, pad], axis=1)


def _tree_sum(vs):
    while len(vs) > 1:
        nxt = [vs[i] + vs[i + 1] for i in range(0, len(vs) - 1, 2)]
        if len(vs) % 2:
            nxt.append(vs[-1])
        vs = nxt
    return vs[0]


def _tree_min(vs):
    while len(vs) > 1:
        nxt = [jnp.minimum(vs[i], vs[i + 1]) for i in range(0, len(vs) - 1, 2)]
        if len(vs) % 2:
            nxt.append(vs[-1])
        vs = nxt
    return vs[0]


def _tree_max(vs):
    while len(vs) > 1:
        nxt = [jnp.maximum(vs[i], vs[i + 1]) for i in range(0, len(vs) - 1, 2)]
        if len(vs) % 2:
            nxt.append(vs[-1])
        vs = nxt
    return vs[0]


def _route_kernel(tok_per_w, s_hbm, w_hbm, i_hbm, s_v, w_v, i_v, col_m):
    wid = lax.axis_index("s") * 2 + lax.axis_index("c")
    base = wid * tok_per_w
    pltpu.sync_copy(s_hbm.at[pl.ds(base * NE_PAD, tok_per_w * NE_PAD)], s_v)

    lanes = lax.iota(jnp.int32, L)
    neg1 = jnp.full((L,), -1.0, jnp.float32)

    def body(i, carry):
        tok128 = (i * L + lanes) * NE_PAD

        gm = []
        for g in range(N_GROUPS):
            grp = [
                plsc.load_gather(s_v, [tok128 + (g * GROUP_SIZE + j)])
                for j in range(GROUP_SIZE)
            ]
            gm.append(_tree_max(grp))

        gsel8 = []
        for _q in range(TOPK_GROUPS):
            cur = _tree_max(gm)
            gidx = _tree_min([
                jnp.where(gm[g] == cur,
                          jnp.full((L,), g, jnp.int32),
                          jnp.full((L,), N_GROUPS, jnp.int32))
                for g in range(N_GROUPS)
            ])
            gm = [jnp.where(gidx == g, neg1, gm[g]) for g in range(N_GROUPS)]
            gsel8.append(gidx * GROUP_SIZE)

        a, b, c, d = gsel8
        a, b = jnp.minimum(a, b), jnp.maximum(a, b)
        c, d = jnp.minimum(c, d), jnp.maximum(c, d)
        a, c = jnp.minimum(a, c), jnp.maximum(a, c)
        b, d = jnp.minimum(b, d), jnp.maximum(b, d)
        b, c = jnp.minimum(b, c), jnp.maximum(b, c)

        cands = []
        for e, gbase in enumerate((a, b, c, d)):
            for j in range(GROUP_SIZE):
                col = gbase + j
                cands.append(plsc.load_gather(s_v, [tok128 + col]))
                col_m[pl.ds((e * GROUP_SIZE + j) * L, L)] = col

        ncand = len(cands)
        bigpos = jnp.full((L,), ncand, jnp.int32)
        ws = []
        idxs = []
        for _k in range(TOPK):
            cur = _tree_max(cands)
            pos = _tree_min([
                jnp.where(cands[e] == cur,
                          jnp.full((L,), e, jnp.int32), bigpos)
                for e in range(ncand)
            ])
            idxs.append(plsc.load_gather(col_m, [pos * L + lanes]))
            cands = [
                jnp.where(pos == e, neg1, cands[e]) for e in range(ncand)
            ]
            ws.append(cur)

        total = _tree_sum(ws)
        scale = ROUTE_SCALE / total
        tok8 = (i * L + lanes) * TOPK
        for k in range(TOPK):
            plsc.store_scatter(w_v, [tok8 + k], ws[k] * scale)
            plsc.store_scatter(i_v, [tok8 + k], idxs[k])
        return carry

    lax.fori_loop(0, tok_per_w // L, body, 0)

    pltpu.sync_copy(w_v, w_hbm.at[pl.ds(base * TOPK, tok_per_w * TOPK)])
    pltpu.sync_copy(i_v, i_hbm.at[pl.ds(base * TOPK, tok_per_w * TOPK)])


@functools.cache
def _make_route(tokc):
    tok_per_w = tokc // NW
    return pl.kernel(
        functools.partial(_route_kernel, tok_per_w),
        mesh=plsc.VectorSubcoreMesh(core_axis_name="c", subcore_axis_name="s"),
        out_type=[
            jax.ShapeDtypeStruct((tokc * TOPK,), jnp.float32),
            jax.ShapeDtypeStruct((tokc * TOPK,), jnp.int32),
        ],
        scratch_types=[
            pltpu.VMEM((tok_per_w * NE_PAD,), jnp.float32),
            pltpu.VMEM((tok_per_w * TOPK,), jnp.float32),
            pltpu.VMEM((tok_per_w * TOPK,), jnp.int32),
            pltpu.VMEM((TOPK_GROUPS * GROUP_SIZE * L,), jnp.int32),
        ],
        compiler_params=pltpu.CompilerParams(needs_layout_passes=False),
    )


@jax.jit
def kernel(x, weight):
    n = x.shape[0]
    w_parts = []
    i_parts = []
    tok_off = 0
    for tokc in CHUNKS:
        off = tok_off // BT
        scores = pl.pallas_call(
            _score_kernel,
            grid=(tokc // BT,),
            in_specs=[
                pl.BlockSpec((BT, DIM), lambda i, off=off: (off + i, 0)),
                pl.BlockSpec((N_EXPERTS, DIM), lambda i: (0, 0)),
            ],
            out_specs=pl.BlockSpec((BT, NE_PAD), lambda i: (i, 0)),
            out_shape=jax.ShapeDtypeStruct((tokc, NE_PAD), jnp.float32),
        )(x, weight)
        w_flat, i_flat = _make_route(tokc)(scores.reshape(-1))
        w_parts.append(w_flat.reshape(tokc, TOPK))
        i_parts.append(i_flat.reshape(tokc, TOPK))
        tok_off += tokc
    return (jnp.concatenate(w_parts, axis=0),
            jnp.concatenate(i_parts, axis=0))
